# baseline (device time: 94204 ns/iter reference)
import jax
import jax.numpy as jnp
from jax import lax
from jax.experimental import pallas as pl
from jax.experimental.pallas import tpu as pltpu

N_DEV = 4


def kernel(x, w_mat, scale_x, scale_w):
    m_per, k = x.shape
    n = w_mat.shape[1]
    m_half = m_per // 2

    def body(x_ref, w_ref, sx_ref, sw_ref, out_ref,
             buf_ref, w8_ref, send_sems, recv_sems):
        my = lax.axis_index("i")
        left = (my - 1) % N_DEV
        right = (my + 1) % N_DEV
        diag = (my + 2) % N_DEV

        buf_ref[0, :, :] = x_ref[:m_half, :].astype(jnp.float8_e4m3fn)
        buf_ref[1, :, :] = x_ref[m_half:, :].astype(jnp.float8_e4m3fn)

        barrier_sem = pltpu.get_barrier_semaphore()
        for nbr in [left, right]:
            pl.semaphore_signal(
                barrier_sem, inc=1,
                device_id=(nbr,), device_id_type=pl.DeviceIdType.MESH,
            )
        pl.semaphore_wait(barrier_sem, 2)

        s = sx_ref[0] * sw_ref[0]

        def copy(src_slot, dst_slot, send_idx, target):
            return pltpu.make_async_remote_copy(
                src_ref=buf_ref.at[src_slot],
                dst_ref=buf_ref.at[dst_slot],
                send_sem=send_sems.at[send_idx],
                recv_sem=recv_sems.at[dst_slot - 2],
                device_id=(target,),
                device_id_type=pl.DeviceIdType.MESH,
            )

        s_cw_a = copy(0, 2, 0, right)
        s_cw_a.start()
        s_ccw_b = copy(1, 5, 1, left)
        s_ccw_b.start()
        s_cw_b = copy(1, 3, 2, right)
        s_cw_b.start()
        s_ccw_a = copy(0, 4, 3, left)
        s_ccw_a.start()

        def recv(dst_slot):
            return copy(0, dst_slot, 0, left)

        w8_ref[...] = w_ref[...].astype(jnp.float8_e5m2)

        def dot_epi_store(slot, origin, half):
            acc = jnp.dot(
                buf_ref[slot, :, :], w8_ref[...],
                preferred_element_type=jnp.float32,
            )
            y = acc * s
            out_ref[pl.ds(origin * m_per + half * m_half, m_half), :] = (
                y / (1.0 + jnp.exp(-y))
            )

        dot_epi_store(0, my, 0)
        dot_epi_store(1, my, 1)

        r_left_a = recv(2)
        r_left_a.wait_recv()
        relay_cw = copy(2, 6, 4, right)
        relay_cw.start()
        r_right_b = recv(5)
        r_right_b.wait_recv()
        relay_ccw = copy(5, 7, 5, left)
        relay_ccw.start()

        dot_epi_store(2, left, 0)
        dot_epi_store(5, right, 1)

        recv(3).wait_recv()
        recv(4).wait_recv()
        dot_epi_store(3, left, 1)
        dot_epi_store(4, right, 0)

        recv(6).wait_recv()
        recv(7).wait_recv()
        dot_epi_store(6, diag, 0)
        dot_epi_store(7, diag, 1)

        for r in [s_cw_a, s_ccw_b, s_cw_b, s_ccw_a, relay_cw, relay_ccw]:
            r.wait_send()

    return pl.pallas_call(
        body,
        out_shape=jax.ShapeDtypeStruct((N_DEV * m_per, n), jnp.float32),
        in_specs=[
            pl.BlockSpec(memory_space=pltpu.VMEM),
            pl.BlockSpec(memory_space=pltpu.VMEM),
            pl.BlockSpec(memory_space=pltpu.SMEM),
            pl.BlockSpec(memory_space=pltpu.SMEM),
        ],
        out_specs=pl.BlockSpec(memory_space=pltpu.VMEM),
        scratch_shapes=[
            pltpu.VMEM((8, m_half, k), jnp.float8_e4m3fn),
            pltpu.VMEM((k, n), jnp.float8_e5m2),
            pltpu.SemaphoreType.DMA((6,)),
            pltpu.SemaphoreType.DMA((6,)),
        ],
        compiler_params=pltpu.CompilerParams(
            collective_id=0,
            vmem_limit_bytes=100 * 1024 * 1024,
        ),
    )(x, w_mat, scale_x, scale_w)


# device time: 92821 ns/iter; 1.0149x vs baseline; 1.0149x over previous
import jax
import jax.numpy as jnp
from jax import lax
from jax.experimental import pallas as pl
from jax.experimental.pallas import tpu as pltpu

N_DEV = 4


def kernel(x, w_mat, scale_x, scale_w):
    m_per, k = x.shape
    n = w_mat.shape[1]
    m_half = m_per // 2

    def body(x_ref, w_ref, sx_ref, sw_ref, out_ref,
             buf_ref, w8_ref, send_sems, recv_sems):
        my = lax.axis_index("i")
        left = (my - 1) % N_DEV
        right = (my + 1) % N_DEV
        diag = (my + 2) % N_DEV

        buf_ref[0, :, :] = x_ref[:m_half, :].astype(jnp.float8_e4m3fn)
        buf_ref[1, :, :] = x_ref[m_half:, :].astype(jnp.float8_e4m3fn)

        barrier_sem = pltpu.get_barrier_semaphore()
        for nbr in [left, right]:
            pl.semaphore_signal(
                barrier_sem, inc=1,
                device_id=(nbr,), device_id_type=pl.DeviceIdType.MESH,
            )
        pl.semaphore_wait(barrier_sem, 2)

        s = sx_ref[0] * sw_ref[0]

        def copy(src_slot, dst_slot, send_idx, recv_idx, target, rows=None):
            if rows is None:
                src = buf_ref.at[src_slot]
                dst = buf_ref.at[dst_slot]
            else:
                src = buf_ref.at[src_slot, pl.ds(rows[0], rows[1]), :]
                dst = buf_ref.at[dst_slot, pl.ds(rows[0], rows[1]), :]
            return pltpu.make_async_remote_copy(
                src_ref=src,
                dst_ref=dst,
                send_sem=send_sems.at[send_idx],
                recv_sem=recv_sems.at[recv_idx],
                device_id=(target,),
                device_id_type=pl.DeviceIdType.MESH,
            )

        s_cw_a = copy(0, 2, 0, 0, right)
        s_cw_a.start()
        s_ccw_b = copy(1, 5, 1, 3, left)
        s_ccw_b.start()
        s_cw_b = copy(1, 3, 2, 1, right)
        s_cw_b.start()
        s_ccw_a = copy(0, 4, 3, 2, left)
        s_ccw_a.start()

        def recv(dst_slot, recv_idx, rows=None):
            return copy(0, dst_slot, 0, recv_idx, left, rows)

        w8_ref[...] = w_ref[...].astype(jnp.float8_e5m2)

        def dot_epi_store(slot, origin, half, rows=(0, None)):
            r0, nr = rows[0], (rows[1] if rows[1] is not None else m_half)
            acc = jnp.dot(
                buf_ref[slot, pl.ds(r0, nr), :], w8_ref[...],
                preferred_element_type=jnp.float32,
            )
            y = acc * s
            row0 = origin * m_per + half * m_half + r0
            out_ref[pl.ds(row0, nr), :] = y / (1.0 + jnp.exp(-y))

        dot_epi_store(0, my, 0)
        dot_epi_store(1, my, 1)

        m_q = m_half // 2
        recv(2, 0).wait_recv()
        relay_cw_0 = copy(2, 6, 4, 4, right, rows=(0, m_q))
        relay_cw_0.start()
        relay_cw_1 = copy(2, 6, 5, 5, right, rows=(m_q, m_q))
        relay_cw_1.start()
        recv(5, 3).wait_recv()
        relay_ccw_0 = copy(5, 7, 6, 6, left, rows=(0, m_q))
        relay_ccw_0.start()
        relay_ccw_1 = copy(5, 7, 7, 7, left, rows=(m_q, m_q))
        relay_ccw_1.start()

        dot_epi_store(2, left, 0)
        dot_epi_store(5, right, 1)

        recv(3, 1).wait_recv()
        recv(4, 2).wait_recv()
        dot_epi_store(3, left, 1)
        dot_epi_store(4, right, 0)

        recv(6, 4, rows=(0, m_q)).wait_recv()
        recv(7, 6, rows=(0, m_q)).wait_recv()
        dot_epi_store(6, diag, 0, rows=(0, m_q))
        dot_epi_store(7, diag, 1, rows=(0, m_q))
        recv(6, 5, rows=(m_q, m_q)).wait_recv()
        recv(7, 7, rows=(m_q, m_q)).wait_recv()
        dot_epi_store(6, diag, 0, rows=(m_q, m_q))
        dot_epi_store(7, diag, 1, rows=(m_q, m_q))

        for r in [s_cw_a, s_ccw_b, s_cw_b, s_ccw_a,
                  relay_cw_0, relay_cw_1, relay_ccw_0, relay_ccw_1]:
            r.wait_send()

    return pl.pallas_call(
        body,
        out_shape=jax.ShapeDtypeStruct((N_DEV * m_per, n), jnp.float32),
        in_specs=[
            pl.BlockSpec(memory_space=pltpu.VMEM),
            pl.BlockSpec(memory_space=pltpu.VMEM),
            pl.BlockSpec(memory_space=pltpu.SMEM),
            pl.BlockSpec(memory_space=pltpu.SMEM),
        ],
        out_specs=pl.BlockSpec(memory_space=pltpu.VMEM),
        scratch_shapes=[
            pltpu.VMEM((8, m_half, k), jnp.float8_e4m3fn),
            pltpu.VMEM((k, n), jnp.float8_e5m2),
            pltpu.SemaphoreType.DMA((8,)),
            pltpu.SemaphoreType.DMA((8,)),
        ],
        compiler_params=pltpu.CompilerParams(
            collective_id=0,
            vmem_limit_bytes=100 * 1024 * 1024,
        ),
    )(x, w_mat, scale_x, scale_w)
